# baseline (device time: 94416 ns/iter reference)
import jax
import jax.numpy as jnp
from jax import lax
from jax.experimental import pallas as pl
from jax.experimental.pallas import tpu as pltpu

N_DEV = 8
M_BLK = 512
K_BLK = 512
BN = 512
N_STEPS = 16
HALF = 4


def kernel(x, w_mat):
    m_tot, k_loc = x.shape
    k_tot, n = w_mat.shape
    assert m_tot == N_DEV * M_BLK and k_loc == K_BLK and k_tot == N_DEV * K_BLK
    xb = x.astype(jnp.bfloat16)
    my_out = lax.axis_index("i")
    s_list = jnp.mod(my_out - jnp.arange(N_DEV, dtype=jnp.int32), N_DEV)

    def body(s_ref, x_ref, w0_ref, w1_ref, w2_ref, w3_ref, out_ref,
             gath_ref, acc_ref, send_sems, recv_sems, cp_sem):
        tp = pl.program_id(0)
        tn = pl.program_id(1)
        my = lax.axis_index("i")
        w_refs = [w0_ref, w1_ref, w2_ref, w3_ref]

        def send(off):
            d = lax.rem(my + off, N_DEV)
            pltpu.make_async_remote_copy(
                src_ref=x_ref.at[pl.ds(d * M_BLK, M_BLK), :],
                dst_ref=gath_ref.at[:, pl.ds(off * K_BLK, K_BLK)],
                send_sem=send_sems.at[off],
                recv_sem=recv_sems.at[off],
                device_id=(d,),
                device_id_type=pl.DeviceIdType.MESH,
            ).start()

        def recv_desc(p):
            return pltpu.make_async_remote_copy(
                src_ref=x_ref.at[pl.ds(0, M_BLK), :],
                dst_ref=gath_ref.at[:, pl.ds(p * K_BLK, K_BLK)],
                send_sem=send_sems.at[0],
                recv_sem=recv_sems.at[p],
                device_id=(my,),
                device_id_type=pl.DeviceIdType.MESH,
            )

        @pl.when((tp == 0) & (tn == 0))
        def _start():
            bar = pltpu.get_barrier_semaphore()
            for off in range(1, N_DEV):
                d = lax.rem(my + off, N_DEV)
                pl.semaphore_signal(
                    bar, inc=1, device_id=(d,),
                    device_id_type=pl.DeviceIdType.MESH,
                )
            pl.semaphore_wait(bar, N_DEV - 1)

            for off in range(1, HALF):
                send(off)
            pltpu.make_async_copy(
                x_ref.at[pl.ds(my * M_BLK, M_BLK), :],
                gath_ref.at[:, pl.ds(0, K_BLK)],
                cp_sem,
            ).start()
            pltpu.make_async_copy(
                x_ref.at[pl.ds(my * M_BLK, M_BLK), :],
                gath_ref.at[:, pl.ds(0, K_BLK)],
                cp_sem,
            ).wait()

            for p in range(1, HALF):
                recv_desc(p).wait_recv()

        @pl.when((tp == 0) & (tn == 1))
        def _second_half_sends():
            for off in range(HALF, N_DEV):
                send(off)

        @pl.when((tp == 1) & (tn == 0))
        def _second_half_recvs():
            for p in range(HALF, N_DEV):
                recv_desc(p).wait_recv()

        y = None
        for q in range(HALF):
            part = jnp.dot(
                gath_ref[:, pl.ds((tp * HALF + q) * K_BLK, K_BLK)],
                w_refs[q][...].astype(jnp.bfloat16),
                preferred_element_type=jnp.float32,
            )
            y = part if y is None else y + part

        @pl.when(tp == 0)
        def _store_partial():
            acc_ref[:, pl.ds(tn * BN, BN)] = y

        @pl.when(tp == 1)
        def _store_final():
            z = acc_ref[:, pl.ds(tn * BN, BN)] + y
            out_ref[...] = (z * jax.nn.sigmoid(z)).astype(jnp.bfloat16)

        @pl.when((tp == 1) & (tn == N_STEPS - 1))
        def _drain():
            for off in range(1, N_DEV):
                pltpu.make_async_remote_copy(
                    src_ref=x_ref.at[pl.ds(0, M_BLK), :],
                    dst_ref=gath_ref.at[:, pl.ds(0, K_BLK)],
                    send_sem=send_sems.at[off],
                    recv_sem=recv_sems.at[0],
                    device_id=(my,),
                    device_id_type=pl.DeviceIdType.MESH,
                ).wait_send()

    def w_spec(q):
        return pl.BlockSpec(
            (K_BLK, BN),
            lambda tp, tn, s_ref, q=q: (s_ref[tp * HALF + q], tn),
        )

    grid_spec = pltpu.PrefetchScalarGridSpec(
        num_scalar_prefetch=1,
        grid=(2, N_STEPS),
        in_specs=[
            pl.BlockSpec((m_tot, K_BLK), lambda tp, tn, s_ref: (0, 0)),
            w_spec(0),
            w_spec(1),
            w_spec(2),
            w_spec(3),
        ],
        out_specs=pl.BlockSpec(
            (M_BLK, BN),
            lambda tp, tn, s_ref: (0, jnp.where(tp == 1, tn, 0)),
        ),
        scratch_shapes=[
            pltpu.VMEM((M_BLK, k_tot), jnp.bfloat16),
            pltpu.VMEM((M_BLK, n), jnp.float32),
            pltpu.SemaphoreType.DMA((N_DEV,)),
            pltpu.SemaphoreType.DMA((N_DEV,)),
            pltpu.SemaphoreType.DMA,
        ],
    )

    return pl.pallas_call(
        body,
        grid_spec=grid_spec,
        out_shape=jax.ShapeDtypeStruct((M_BLK, n), jnp.bfloat16),
        compiler_params=pltpu.CompilerParams(
            dimension_semantics=("arbitrary", "arbitrary"),
            collective_id=0,
        ),
    )(s_list, xb, w_mat, w_mat, w_mat, w_mat)
